# baseline (speedup 1.0000x reference)
import jax
import jax.numpy as jnp
from jax.experimental import pallas as pl
from jax.experimental.pallas import tpu as pltpu

_SIZES = (1024, 1024, 1024, 1024, 1024, 1024, 1024, 512, 256, 256)
_BMAX = max(_SIZES)
_NBUF = 3


def _gate_kernel(x_hbm, w_ref, o_hbm, buf, obuf, isem, osem):
    steps = len(_SIZES)
    offs = [sum(_SIZES[:i]) for i in range(steps)]

    def _in_copy(i):
        slot, size, off = i % _NBUF, _SIZES[i], offs[i]
        return pltpu.make_async_copy(
            x_hbm.at[pl.ds(off, size), :], buf.at[slot, pl.ds(0, size), :],
            isem.at[slot])

    def _out_copy(i):
        slot, size, off = i % _NBUF, _SIZES[i], offs[i]
        return pltpu.make_async_copy(
            obuf.at[slot, pl.ds(0, size), :], o_hbm.at[pl.ds(off, size), :],
            osem.at[slot])

    for i in range(_NBUF - 1):
        _in_copy(i).start()

    for i in range(steps):
        if i + _NBUF - 1 < steps:
            _in_copy(i + _NBUF - 1).start()
        _in_copy(i).wait()
        if i >= _NBUF:
            _out_copy(i - _NBUF).wait()
        slot, size = i % _NBUF, _SIZES[i]
        obuf[slot, pl.ds(0, size), :] = jax.lax.dot_general(
            buf[slot, pl.ds(0, size), :], w_ref[...],
            dimension_numbers=(((1,), (1,)), ((), ())),
            preferred_element_type=jnp.float32,
        )
        _out_copy(i).start()

    for i in range(steps - _NBUF, steps):
        _out_copy(i).wait()


def kernel(hidden_states, weight):
    m, k = hidden_states.shape
    e = weight.shape[0]
    return pl.pallas_call(
        _gate_kernel,
        in_specs=[
            pl.BlockSpec(memory_space=pltpu.HBM),
            pl.BlockSpec(memory_space=pltpu.VMEM),
        ],
        out_specs=pl.BlockSpec(memory_space=pltpu.HBM),
        out_shape=jax.ShapeDtypeStruct((m, e), jnp.float32),
        scratch_shapes=[
            pltpu.VMEM((_NBUF, _BMAX, k), jnp.float32),
            pltpu.VMEM((_NBUF, _BMAX, e), jnp.float32),
            pltpu.SemaphoreType.DMA((_NBUF,)),
            pltpu.SemaphoreType.DMA((_NBUF,)),
        ],
    )(hidden_states, weight)


# speedup vs baseline: 1.0928x; 1.0928x over previous
import jax
import jax.numpy as jnp
from jax.experimental import pallas as pl
from jax.experimental.pallas import tpu as pltpu

_BM = 1024


def _gate_kernel(x_ref, w_ref, o_ref):
    o_ref[...] = jax.lax.dot_general(
        x_ref[...], w_ref[...],
        dimension_numbers=(((1,), (1,)), ((), ())),
        preferred_element_type=jnp.float32,
        precision=jax.lax.Precision.DEFAULT,
    )


def kernel(hidden_states, weight):
    m, k = hidden_states.shape
    e = weight.shape[0]
    return pl.pallas_call(
        _gate_kernel,
        grid=(m // _BM,),
        in_specs=[
            pl.BlockSpec((_BM, k), lambda i: (i, 0)),
            pl.BlockSpec((e, k), lambda i: (0, 0)),
        ],
        out_specs=pl.BlockSpec((_BM, e), lambda i: (i, 0)),
        out_shape=jax.ShapeDtypeStruct((m, e), jnp.float32),
        compiler_params=pltpu.CompilerParams(
            dimension_semantics=("arbitrary",),
        ),
    )(hidden_states, weight)
